# speedup vs baseline: 16.4686x; 16.4686x over previous
import functools

import jax
import jax.numpy as jnp
from jax.experimental import pallas as pl

_CONF_TH = 0.85
_IGNORE = 2
_EPS = 1e-07
_DROP = 0.5

_ROWS = 2048
_COLS = 128
_N = _ROWS * _COLS

_LO0 = 0x3F3FFFFF
_HI0 = 0x41000000


def _body(pred_ref, mask_ref, train_ref, hold_ref, part_ref):
    p = pred_ref[0]
    m = mask_ref[0]
    ann = m != _IGNORE
    mf = m.astype(jnp.float32)
    conf = jnp.maximum(p, 1.0 - p)
    corr = (p > 0.5) == (mf == 1.0)
    isconf = conf > _CONF_TH

    score = jnp.ones_like(p)
    score = jnp.where(isconf & corr, 1.0, score)
    score = jnp.where((~isconf) & corr, 2.0, score)
    score = jnp.where((~isconf) & (~corr), 3.0, score)
    score = jnp.where(isconf & (~corr), 4.0, score)
    bonus = (conf - 0.5) * 0.5
    s = jnp.where(corr, score - bonus, score + bonus)

    key = jnp.where(ann, jax.lax.bitcast_convert_type(s, jnp.int32), 0)

    npts = jnp.sum(ann.astype(jnp.int32))
    k_train = jnp.floor(npts.astype(jnp.float32) * (1.0 - _DROP)).astype(jnp.int32)

    def step1(_, carry):
        lo, hi = carry
        mid = lo + (hi - lo) // 2
        c = jnp.sum((key > mid).astype(jnp.int32))
        big = c >= k_train
        return jnp.where(big, mid, lo), jnp.where(big, hi, mid)

    lo, hi = jax.lax.fori_loop(0, 25, step1, (jnp.int32(_LO0), jnp.int32(_HI0)))
    t_key = hi
    n_greater = jnp.sum((key > t_key).astype(jnp.int32))
    r = k_train - n_greater

    rows = jax.lax.broadcasted_iota(jnp.int32, (_ROWS, _COLS), 0)
    cols = jax.lax.broadcasted_iota(jnp.int32, (_ROWS, _COLS), 1)
    fi = rows * _COLS + cols
    eq = key == t_key

    def step2(_, carry):
        lo2, hi2 = carry
        mid = lo2 + (hi2 - lo2) // 2
        c = jnp.sum((eq & (fi < mid)).astype(jnp.int32))
        geq = c >= r
        return jnp.where(geq, lo2, mid), jnp.where(geq, mid, hi2)

    _, m_star = jax.lax.fori_loop(0, 19, step2, (jnp.int32(0), jnp.int32(_N)))

    train = (key > t_key) | (eq & (fi < m_star))
    hold = ann & (~train)

    pcl = jnp.clip(p, _EPS, 1.0 - _EPS)
    bce = -(mf * jnp.log(pcl) + (1.0 - mf) * jnp.log(1.0 - pcl))
    tf32 = train.astype(jnp.float32)
    hf32 = hold.astype(jnp.float32)
    num = jnp.sum(bce * tf32)
    den = jnp.sum(tf32)

    cc = (isconf & corr).astype(jnp.float32)
    ci = (isconf & (~corr)).astype(jnp.float32)
    uc = ((~isconf) & corr).astype(jnp.float32)
    ui = ((~isconf) & (~corr)).astype(jnp.float32)

    vals = [
        num,
        den,
        jnp.sum(cc * tf32),
        jnp.sum(ci * tf32),
        jnp.sum(uc * tf32),
        jnp.sum(ui * tf32),
        den,
        jnp.sum(cc * hf32),
        jnp.sum(ci * hf32),
        jnp.sum(uc * hf32),
        jnp.sum(ui * hf32),
        jnp.sum(hf32),
    ]
    col = jax.lax.broadcasted_iota(jnp.int32, (1, _COLS), 1)
    out = jnp.zeros((1, _COLS), jnp.float32)
    for j, v in enumerate(vals):
        out = jnp.where(col == j, v, out)
    part_ref[0] = out

    train_ref[0] = train.astype(jnp.int8)
    hold_ref[0] = hold.astype(jnp.int8)


@jax.jit
def kernel(pred, mask):
    if pred.ndim == 4 and pred.shape[1] == 1:
        pred = pred[:, 0]
    b = pred.shape[0]
    pred3 = pred.reshape(b, _ROWS, _COLS)
    mask3 = mask.astype(jnp.int32).reshape(b, _ROWS, _COLS)

    train8, hold8, parts = pl.pallas_call(
        _body,
        grid=(b,),
        in_specs=[
            pl.BlockSpec((1, _ROWS, _COLS), lambda i: (i, 0, 0)),
            pl.BlockSpec((1, _ROWS, _COLS), lambda i: (i, 0, 0)),
        ],
        out_specs=[
            pl.BlockSpec((1, _ROWS, _COLS), lambda i: (i, 0, 0)),
            pl.BlockSpec((1, _ROWS, _COLS), lambda i: (i, 0, 0)),
            pl.BlockSpec((1, 1, _COLS), lambda i: (i, 0, 0)),
        ],
        out_shape=[
            jax.ShapeDtypeStruct((b, _ROWS, _COLS), jnp.int8),
            jax.ShapeDtypeStruct((b, _ROWS, _COLS), jnp.int8),
            jax.ShapeDtypeStruct((b, 1, _COLS), jnp.float32),
        ],
    )(pred3, mask3)

    train = train8.reshape(b, 512, 512).astype(bool)
    hold = hold8.reshape(b, 512, 512).astype(bool)
    parts = parts[:, 0, :]
    num = parts[:, 0].sum()
    den = parts[:, 1].sum()
    loss = num / (den + _EPS)
    stats10 = parts[:, 2:12].sum(axis=0)
    n_holdout = stats10[9]
    n_h_correct = stats10[5] + stats10[7]
    acc = jnp.where(
        n_holdout > 0, n_h_correct / jnp.maximum(n_holdout, 1.0), 0.0
    ).astype(jnp.float32)
    stats = jnp.concatenate([stats10, acc[None]])
    return loss, train, hold, stats
